# baseline (speedup 1.0000x reference)
import jax
import jax.numpy as jnp
from jax.experimental import pallas as pl
from jax.experimental.pallas import tpu as pltpu

N = 50000
E = 800000
D = 100
NUM_REL = 3


def _leaky(x):
    return jnp.where(x > 0, x, 0.01 * x)


def _batchnorm(x, g, b, eps=1e-5):
    m = jnp.mean(x, axis=0)
    v = jnp.var(x, axis=0)
    return (x - m) / jnp.sqrt(v + eps) * g + b


def _rgcn(x, edge_index, edge_type, W, root, bias):
    src = edge_index[0]
    dst = edge_index[1]
    n = x.shape[0]
    out = x @ root + bias
    for r in range(NUM_REL):
        mask = (edge_type == r).astype(x.dtype)
        msg = x[src] * mask[:, None]
        agg = jnp.zeros((n, x.shape[1]), x.dtype).at[dst].add(msg)
        cnt = jnp.zeros((n,), x.dtype).at[dst].add(mask)
        agg = agg / jnp.maximum(cnt, 1.0)[:, None]
        out = out + agg @ W[r]
    return out


def _head_kernel(x_ref, w1_ref, b1_ref, w2_ref, b2_ref, w3_ref, b3_ref, o_ref):
    x = x_ref[...]
    h = _leaky(jnp.dot(x, w1_ref[...], preferred_element_type=jnp.float32) + b1_ref[...])
    h = _leaky(jnp.dot(h, w2_ref[...], preferred_element_type=jnp.float32) + b2_ref[...])
    o_ref[...] = jnp.dot(h, w3_ref[...], preferred_element_type=jnp.float32) + b3_ref[...]


def _head(x, p):
    BLK = 1000
    grid = (N // BLK,)
    w3 = jnp.zeros((D, 128), jnp.float32).at[:, :2].set(p['W_o3'])
    b3 = jnp.zeros((128,), jnp.float32).at[:2].set(p['b_o3'])
    out = pl.pallas_call(
        _head_kernel,
        grid=grid,
        in_specs=[
            pl.BlockSpec((BLK, D), lambda i: (i, 0)),
            pl.BlockSpec((D, D), lambda i: (0, 0)),
            pl.BlockSpec((D,), lambda i: (0,)),
            pl.BlockSpec((D, D), lambda i: (0, 0)),
            pl.BlockSpec((D,), lambda i: (0,)),
            pl.BlockSpec((D, 128), lambda i: (0, 0)),
            pl.BlockSpec((128,), lambda i: (0,)),
        ],
        out_specs=pl.BlockSpec((BLK, 128), lambda i: (i, 0)),
        out_shape=jax.ShapeDtypeStruct((N, 128), jnp.float32),
    )(x, p['W_o1'], p['b_o1'], p['W_o2'], p['b_o2'], w3, b3)
    return out[:, :2]


def kernel(num_prop, cat_prop, tweet_emb, user_emb, user_name_emb, edge_index, edge_type, params):
    p = params
    n = _leaky(num_prop @ p['W_num'] + p['b_num'])
    c = _leaky(cat_prop @ p['W_cat'] + p['b_cat'])
    t = _leaky(tweet_emb @ p['W_tweet'] + p['b_tweet'])
    u = _leaky(user_emb @ p['W_user'] + p['b_user'])
    un = _leaky(user_name_emb @ p['W_uname'] + p['b_uname'])
    x = jnp.concatenate([n, c, t, u, un], axis=1)
    x = _leaky(x @ p['W_in'] + p['b_in'])
    x = _rgcn(x, edge_index, edge_type, p['rgcn1_W'], p['rgcn1_root'], p['rgcn1_b'])
    x = _leaky(_batchnorm(x, p['bn1_g'], p['bn1_b']))
    x = _rgcn(x, edge_index, edge_type, p['rgcn2_W'], p['rgcn2_root'], p['rgcn2_b'])
    x = _leaky(_batchnorm(x, p['bn2_g'], p['bn2_b']))
    return _head(x, p)


# speedup vs baseline: 4.1802x; 4.1802x over previous
import jax
import jax.numpy as jnp
from jax import lax
from jax.experimental import pallas as pl
from jax.experimental.pallas import tpu as pltpu
from jax.experimental.pallas import tpu_sc as plsc

N = 50000
E = 800000
D = 100
NUM_REL = 3
DP = 128
ONES_LO = 112

CDST = 2560
NCHUNK = 20
NPAD = CDST * NCHUNK
ACC_ROWS = NUM_REL * CDST + 16
EPT = E // 16
EB = 2000
LCAP = 8192
K = 128
LROWS = LCAP // K

HI = lax.Precision.HIGHEST


def _leaky(x):
    return jnp.where(x > 0, x, 0.01 * x)


def _enc_kernel(num_ref, cat_ref, tw_ref, us_ref, un_ref,
                wn_ref, wc_ref, wt_ref, wu_ref, wun_ref, pb_ref,
                win_ref, ones_ref, x_ref):
    f32 = jnp.float32
    acc = ones_ref[1][None, :]
    for j, (e_ref, w_ref) in enumerate(((num_ref, wn_ref), (cat_ref, wc_ref),
                                        (tw_ref, wt_ref), (us_ref, wu_ref),
                                        (un_ref, wun_ref))):
        part = _leaky(jnp.dot(e_ref[...], w_ref[...],
                              preferred_element_type=f32, precision=HI)
                      + pb_ref[j])
        acc = acc + jnp.dot(part, win_ref[j], preferred_element_type=f32,
                            precision=HI)
    x0 = _leaky(acc)
    ones = ones_ref[0][None, :]
    x_ref[...] = x0 * (1.0 - ones) + ones


def _combine_kernel(x_ref, agg_ref, root_ref, wrel_ref, bias_ref,
                    h_ref, stats_ref, acc_ref):
    f32 = jnp.float32
    i = pl.program_id(0)
    h = jnp.dot(x_ref[...], root_ref[...], preferred_element_type=f32, precision=HI) \
        + bias_ref[0]
    for r in range(NUM_REL):
        blk = agg_ref[r]
        cnt = jnp.max(blk[:, ONES_LO:], axis=1, keepdims=True)
        mean = blk / jnp.maximum(cnt, 1.0)
        h = h + jnp.dot(mean, wrel_ref[r], preferred_element_type=f32, precision=HI)
    h_ref[...] = h

    @pl.when(i == 0)
    def _():
        acc_ref[...] = jnp.zeros_like(acc_ref)

    acc_ref[0, :] += jnp.sum(h, axis=0)
    acc_ref[1, :] += jnp.sum(h * h, axis=0)

    @pl.when(i == pl.num_programs(0) - 1)
    def _():
        stats_ref[...] = acc_ref[...]


def _bn(h, stats_ref, gb_ref):
    mean = stats_ref[0] * (1.0 / N)
    var = stats_ref[1] * (1.0 / N) - mean * mean
    scale = gb_ref[0] / jnp.sqrt(var + 1e-5)
    return _leaky((h - mean) * scale + gb_ref[1])


def _bn_kernel(h_ref, stats_ref, gb_ref, ones_ref, x_ref):
    x = _bn(h_ref[...], stats_ref, gb_ref)
    ones = ones_ref[0][None, :]
    x_ref[...] = x * (1.0 - ones) + ones


def _head_kernel(h_ref, stats_ref, gb_ref, w1_ref, b1_ref, w2_ref, b2_ref,
                 w3_ref, b3_ref, o_ref):
    f32 = jnp.float32
    x = _bn(h_ref[...], stats_ref, gb_ref)
    x = _leaky(jnp.dot(x, w1_ref[...], preferred_element_type=f32, precision=HI) + b1_ref[...])
    x = _leaky(jnp.dot(x, w2_ref[...], preferred_element_type=f32, precision=HI) + b2_ref[...])
    o_ref[...] = jnp.dot(x, w3_ref[...], preferred_element_type=f32, precision=HI) + b3_ref[...]


def _edge_pass_body(tab_hbm, src_hbm, dst_hbm, rel_hbm, zeros_hbm,
                    agg_hbm,
                    e_src, e_dst, e_rel, glist, alist, buf, zbuf,
                    acc, sem):
    cid = lax.axis_index("c")
    sid = lax.axis_index("s")
    e0 = sid * EPT
    trash = ACC_ROWS - 16 + sid
    lanes = lax.iota(jnp.int32, 16)

    pltpu.sync_copy(zeros_hbm, zbuf)

    def flush(j, done):
        row = (done // K) & (LROWS - 1)
        pltpu.async_copy(tab_hbm.at[glist.at[row]], buf, sem).wait()
        pltpu.sync_copy(buf, acc.at[alist.at[row]], add=True)
        return done + K

    def chunk_body(k, _unused):
        base_dst = (cid * (NCHUNK // 2) + k) * CDST
        z0 = sid * (ACC_ROWS // 16)
        for off in range(0, ACC_ROWS // 16, K):
            ln = min(K, ACC_ROWS // 16 - off)
            pltpu.sync_copy(zbuf.at[pl.ds(0, ln)], acc.at[pl.ds(z0 + off, ln)])
        plsc.subcore_barrier()

        def eblock(b, carry):
            off_s, done = carry
            s0 = pl.multiple_of(e0 + b * EB, 8)
            pltpu.sync_copy(src_hbm.at[pl.ds(s0, EB)], e_src)
            pltpu.sync_copy(dst_hbm.at[pl.ds(s0, EB)], e_dst)
            pltpu.sync_copy(rel_hbm.at[pl.ds(s0, EB)], e_rel)

            def vit(i, offv):
                ii = pl.multiple_of(i * 16, 16)
                s = e_src[pl.ds(ii, 16)]
                d = e_dst[pl.ds(ii, 16)]
                r = e_rel[pl.ds(ii, 16)]
                dl = d - base_dst
                m = (dl >= 0) & (dl < CDST)
                ai = r * CDST + dl
                pos = offv + plsc.cumsum(m.astype(jnp.int32)) - 1
                prow = (pos // K) & (LROWS - 1)
                pcol = pos & (K - 1)
                plsc.store_scatter(glist, [prow, pcol], s, mask=m)
                plsc.store_scatter(alist, [prow, pcol], ai, mask=m)
                return offv + plsc.all_reduce_population_count(m)

            offv = lax.fori_loop(0, EB // 16, vit,
                                 jnp.full((16,), off_s, jnp.int32))
            off_s2 = jnp.max(offv)
            nb = (off_s2 - done) // K
            done2 = lax.fori_loop(0, nb, flush, done)
            return off_s2, done2

        off_s, done = lax.fori_loop(0, EPT // EB, eblock,
                                    (jnp.int32(0), jnp.int32(0)))
        rem = off_s - done

        def padfill(i, _):
            pos = off_s + i * 16 + lanes
            m = pos < done + K
            prow = (pos // K) & (LROWS - 1)
            pcol = pos & (K - 1)
            plsc.store_scatter(glist, [prow, pcol],
                               jnp.full((16,), sid * 64, jnp.int32), mask=m)
            plsc.store_scatter(alist, [prow, pcol],
                               jnp.full((16,), trash, jnp.int32), mask=m)
            return 0

        @pl.when(rem > 0)
        def _():
            lax.fori_loop(0, K // 16, padfill, 0)
            flush(0, done)

        plsc.subcore_barrier()
        for r in range(NUM_REL):
            pltpu.sync_copy(
                acc.at[pl.ds(r * CDST + sid * (CDST // 16), CDST // 16)],
                agg_hbm.at[r, pl.ds(base_dst + sid * (CDST // 16), CDST // 16)])
        plsc.subcore_barrier()
        return 0

    lax.fori_loop(0, NCHUNK // 2, chunk_body, 0)


def _edge_pass(table, src, dst, rel, zeros_blk):
    mesh = plsc.VectorSubcoreMesh(core_axis_name="c", subcore_axis_name="s")
    f = pl.kernel(
        _edge_pass_body,
        mesh=mesh,
        compiler_params=pltpu.CompilerParams(needs_layout_passes=False),
        out_type=jax.ShapeDtypeStruct((NUM_REL, NPAD, DP), jnp.float32),
        scratch_types=[
            pltpu.VMEM((EB,), jnp.int32),
            pltpu.VMEM((EB,), jnp.int32),
            pltpu.VMEM((EB,), jnp.int32),
            pltpu.VMEM((LROWS, K), jnp.int32),
            pltpu.VMEM((LROWS, K), jnp.int32),
            pltpu.VMEM((K, DP), jnp.float32),
            pltpu.VMEM((K, DP), jnp.float32),
            pltpu.VMEM_SHARED((ACC_ROWS, DP), jnp.float32),
            pltpu.SemaphoreType.DMA,
        ],
    )
    return f(table, src, dst, rel, zeros_blk)


def _pad2(w, rows, cols=DP):
    out = jnp.zeros((rows, cols), jnp.float32)
    return out.at[:w.shape[0], :w.shape[1]].set(w)


def _pad1(b, cols=DP):
    return jnp.zeros((cols,), jnp.float32).at[:b.shape[0]].set(b)


def kernel(num_prop, cat_prop, tweet_emb, user_emb, user_name_emb,
           edge_index, edge_type, params):
    p = params
    f32 = jnp.float32
    src = edge_index[0].astype(jnp.int32)
    dst = edge_index[1].astype(jnp.int32)
    rel = edge_type.astype(jnp.int32)

    IND = D // 5
    wn = _pad2(p['W_num'], 5)
    wc = _pad2(p['W_cat'], 3)
    wt = _pad2(p['W_tweet'], 768)
    wu = _pad2(p['W_user'], 768)
    wun = _pad2(p['W_uname'], 768)
    pb = jnp.stack([_pad1(p['b_num']), _pad1(p['b_cat']), _pad1(p['b_tweet']),
                    _pad1(p['b_user']), _pad1(p['b_uname'])])
    win_parts = jnp.stack([
        _pad2(p['W_in'][i * IND:(i + 1) * IND, :], DP) for i in range(5)])
    ones1 = jnp.zeros((3, DP), f32)
    ones1 = ones1.at[0, ONES_LO:].set(1.0)
    ones1 = ones1.at[1, :D].set(p['b_in'])

    root1 = _pad2(p['rgcn1_root'], DP)
    w1 = jnp.stack([_pad2(p['rgcn1_W'][r], DP) for r in range(NUM_REL)])
    b1row = _pad1(p['rgcn1_b'])[None, :]
    root2 = _pad2(p['rgcn2_root'], DP)
    w2 = jnp.stack([_pad2(p['rgcn2_W'][r], DP) for r in range(NUM_REL)])
    b2row = _pad1(p['rgcn2_b'])[None, :]

    BLK = 1000
    grid = (N // BLK,)

    def rowspec(d2=DP):
        return pl.BlockSpec((BLK, d2), lambda i: (i, 0))

    def fullspec(shape):
        nd = len(shape)
        return pl.BlockSpec(shape, lambda i: (0,) * nd)

    x0 = pl.pallas_call(
        _enc_kernel,
        grid=grid,
        in_specs=[
            pl.BlockSpec((BLK, 5), lambda i: (i, 0)),
            pl.BlockSpec((BLK, 3), lambda i: (i, 0)),
            pl.BlockSpec((BLK, 768), lambda i: (i, 0)),
            pl.BlockSpec((BLK, 768), lambda i: (i, 0)),
            pl.BlockSpec((BLK, 768), lambda i: (i, 0)),
            fullspec((5, DP)), fullspec((3, DP)), fullspec((768, DP)),
            fullspec((768, DP)), fullspec((768, DP)), fullspec((5, DP)),
            fullspec((5, DP, DP)), fullspec((3, DP)),
        ],
        out_specs=rowspec(),
        out_shape=jax.ShapeDtypeStruct((N, DP), f32),
    )(num_prop, cat_prop, tweet_emb, user_emb, user_name_emb,
      wn, wc, wt, wu, wun, pb, win_parts, ones1)

    zeros_blk = jnp.zeros((K, DP), f32)

    def combine(x_arr, agg, root, wrel, brow):
        return pl.pallas_call(
            _combine_kernel,
            grid=grid,
            in_specs=[rowspec(),
                      pl.BlockSpec((NUM_REL, BLK, DP), lambda i: (0, i, 0)),
                      fullspec((DP, DP)), fullspec((NUM_REL, DP, DP)),
                      fullspec((1, DP))],
            out_specs=[rowspec(), fullspec((2, DP))],
            out_shape=[jax.ShapeDtypeStruct((N, DP), f32),
                       jax.ShapeDtypeStruct((2, DP), f32)],
            scratch_shapes=[pltpu.VMEM((2, DP), f32)],
        )(x_arr, agg, root, wrel, brow)

    def bn_apply(h, stats, gb):
        return pl.pallas_call(
            _bn_kernel,
            grid=grid,
            in_specs=[rowspec(), fullspec((2, DP)), fullspec((2, DP)),
                      fullspec((3, DP))],
            out_specs=rowspec(),
            out_shape=jax.ShapeDtypeStruct((N, DP), f32),
        )(h, stats, gb, ones1)

    agg1 = _edge_pass(x0, src, dst, rel, zeros_blk)
    h1, stats1 = combine(x0, agg1, root1, w1, b1row)
    gb1 = jnp.stack([_pad1(p['bn1_g']), _pad1(p['bn1_b'])])
    x1 = bn_apply(h1, stats1, gb1)

    agg2 = _edge_pass(x1, src, dst, rel, zeros_blk)
    h2, stats2 = combine(x1, agg2, root2, w2, b2row)
    gb2 = jnp.stack([_pad1(p['bn2_g']), _pad1(p['bn2_b'])])

    out = pl.pallas_call(
        _head_kernel,
        grid=grid,
        in_specs=[rowspec(), fullspec((2, DP)), fullspec((2, DP)),
                  fullspec((DP, DP)), fullspec((DP,)),
                  fullspec((DP, DP)), fullspec((DP,)),
                  fullspec((DP, DP)), fullspec((DP,))],
        out_specs=rowspec(),
        out_shape=jax.ShapeDtypeStruct((N, DP), f32),
    )(h2, stats2, gb2, _pad2(p['W_o1'], DP), _pad1(p['b_o1']),
      _pad2(p['W_o2'], DP), _pad1(p['b_o2']), _pad2(p['W_o3'], DP),
      _pad1(p['b_o3']))
    return out[:, :2]


# speedup vs baseline: 5.1422x; 1.2301x over previous
import jax
import jax.numpy as jnp
from jax import lax
from jax.experimental import pallas as pl
from jax.experimental.pallas import tpu as pltpu
from jax.experimental.pallas import tpu_sc as plsc

N = 50000
E = 800000
D = 100
NUM_REL = 3
DP = 128
ONES_LO = 112

CDST = 2560
NCHUNK = 20
NPAD = CDST * NCHUNK
ACC_ROWS = NUM_REL * CDST + 16
EPT = E // 16
EPT_PAD = 51200
EB = 2048
LCAP = 8192
K = 128
LROWS = LCAP // K

HI = lax.Precision.HIGHEST


def _leaky(x):
    return jnp.where(x > 0, x, 0.01 * x)


def _enc_kernel(num_ref, cat_ref, tw_ref, us_ref, un_ref,
                wn_ref, wc_ref, wt_ref, wu_ref, wun_ref, pb_ref,
                win_ref, ones_ref, x_ref):
    f32 = jnp.float32
    acc = ones_ref[1][None, :]
    for j, (e_ref, w_ref) in enumerate(((num_ref, wn_ref), (cat_ref, wc_ref),
                                        (tw_ref, wt_ref), (us_ref, wu_ref),
                                        (un_ref, wun_ref))):
        part = _leaky(jnp.dot(e_ref[...], w_ref[...],
                              preferred_element_type=f32, precision=HI)
                      + pb_ref[j])
        acc = acc + jnp.dot(part, win_ref[j], preferred_element_type=f32,
                            precision=HI)
    x0 = _leaky(acc)
    ones = ones_ref[0][None, :]
    x_ref[...] = x0 * (1.0 - ones) + ones


def _combine_kernel(x_ref, agg_ref, root_ref, wrel_ref, bias_ref,
                    h_ref, stats_ref, acc_ref):
    f32 = jnp.float32
    i = pl.program_id(0)
    h = jnp.dot(x_ref[...], root_ref[...], preferred_element_type=f32, precision=HI) \
        + bias_ref[0]
    for r in range(NUM_REL):
        blk = agg_ref[r]
        cnt = jnp.max(blk[:, ONES_LO:], axis=1, keepdims=True)
        mean = blk / jnp.maximum(cnt, 1.0)
        h = h + jnp.dot(mean, wrel_ref[r], preferred_element_type=f32, precision=HI)
    h_ref[...] = h

    @pl.when(i == 0)
    def _():
        acc_ref[...] = jnp.zeros_like(acc_ref)

    acc_ref[0, :] += jnp.sum(h, axis=0)
    acc_ref[1, :] += jnp.sum(h * h, axis=0)

    @pl.when(i == pl.num_programs(0) - 1)
    def _():
        stats_ref[...] = acc_ref[...]


def _bn(h, stats_ref, gb_ref):
    mean = stats_ref[0] * (1.0 / N)
    var = stats_ref[1] * (1.0 / N) - mean * mean
    scale = gb_ref[0] / jnp.sqrt(var + 1e-5)
    return _leaky((h - mean) * scale + gb_ref[1])


def _bn_kernel(h_ref, stats_ref, gb_ref, ones_ref, x_ref):
    x = _bn(h_ref[...], stats_ref, gb_ref)
    ones = ones_ref[0][None, :]
    x_ref[...] = x * (1.0 - ones) + ones


def _head_kernel(h_ref, stats_ref, gb_ref, w1_ref, b1_ref, w2_ref, b2_ref,
                 w3_ref, b3_ref, o_ref):
    f32 = jnp.float32
    x = _bn(h_ref[...], stats_ref, gb_ref)
    x = _leaky(jnp.dot(x, w1_ref[...], preferred_element_type=f32, precision=HI) + b1_ref[...])
    x = _leaky(jnp.dot(x, w2_ref[...], preferred_element_type=f32, precision=HI) + b2_ref[...])
    o_ref[...] = jnp.dot(x, w3_ref[...], preferred_element_type=f32, precision=HI) + b3_ref[...]


def _edge_pass_body(tab_hbm, edata_hbm, zeros_hbm,
                    agg_hbm,
                    e_a, e_b, glist, alist, buf, zbuf,
                    acc, sem, sem_e):
    cid = lax.axis_index("c")
    sid = lax.axis_index("s")
    e0 = sid * EPT
    trash = ACC_ROWS - 16 + sid
    lanes = lax.iota(jnp.int32, 16)
    NB = EPT_PAD // EB

    pltpu.sync_copy(zeros_hbm, zbuf)

    def stage(b, dst_ref):
        s0 = pl.multiple_of(b * EB, EB)
        return pltpu.async_copy(edata_hbm.at[sid, :, pl.ds(s0, EB)], dst_ref,
                                sem_e)

    def flush(j, done):
        row = (done // K) & (LROWS - 1)
        pltpu.async_copy(tab_hbm.at[glist.at[row]], buf, sem).wait()
        pltpu.sync_copy(buf, acc.at[alist.at[row]], add=True)
        return done + K

    def chunk_body(k, _unused):
        base_dst = (cid * (NCHUNK // 2) + k) * CDST
        z0 = sid * (ACC_ROWS // 16)
        for off in range(0, ACC_ROWS // 16, K):
            ln = min(K, ACC_ROWS // 16 - off)
            pltpu.sync_copy(zbuf.at[pl.ds(0, ln)], acc.at[pl.ds(z0 + off, ln)])
        plsc.subcore_barrier()

        def scan(e_ref, carry):
            off_s, done = carry

            def vit(i, offv):
                ii = pl.multiple_of(i * 16, 16)
                s = e_ref[0, pl.ds(ii, 16)]
                d = e_ref[1, pl.ds(ii, 16)]
                r = e_ref[2, pl.ds(ii, 16)]
                dl = d - base_dst
                m = (dl >= 0) & (dl < CDST)
                ai = r * CDST + dl
                pos = offv + plsc.cumsum(m.astype(jnp.int32)) - 1
                prow = (pos // K) & (LROWS - 1)
                pcol = pos & (K - 1)
                plsc.store_scatter(glist, [prow, pcol], s, mask=m)
                plsc.store_scatter(alist, [prow, pcol], ai, mask=m)
                return offv + plsc.all_reduce_population_count(m)

            offv = lax.fori_loop(0, EB // 16, vit,
                                 jnp.full((16,), off_s, jnp.int32))
            off_s2 = jnp.max(offv)
            nb = (off_s2 - done) // K
            done2 = lax.fori_loop(0, nb, flush, done)
            return off_s2, done2

        stage(0, e_a).wait()

        def pair(j, carry):
            cp_b = stage(2 * j + 1, e_b)
            carry = scan(e_a, carry)
            cp_b.wait()
            cp_a = stage(2 * j + 2, e_a)
            carry = scan(e_b, carry)
            cp_a.wait()
            return carry

        carry = lax.fori_loop(0, (NB - 1) // 2, pair,
                              (jnp.int32(0), jnp.int32(0)))
        off_s, done = scan(e_a, carry)
        rem = off_s - done

        def padfill(i, _):
            pos = off_s + i * 16 + lanes
            m = pos < done + K
            prow = (pos // K) & (LROWS - 1)
            pcol = pos & (K - 1)
            plsc.store_scatter(glist, [prow, pcol],
                               jnp.full((16,), sid * 64, jnp.int32), mask=m)
            plsc.store_scatter(alist, [prow, pcol],
                               jnp.full((16,), trash, jnp.int32), mask=m)
            return 0

        @pl.when(rem > 0)
        def _():
            lax.fori_loop(0, K // 16, padfill, 0)
            flush(0, done)

        plsc.subcore_barrier()
        for r in range(NUM_REL):
            pltpu.sync_copy(
                acc.at[pl.ds(r * CDST + sid * (CDST // 16), CDST // 16)],
                agg_hbm.at[r, pl.ds(base_dst + sid * (CDST // 16), CDST // 16)])
        plsc.subcore_barrier()
        return 0

    lax.fori_loop(0, NCHUNK // 2, chunk_body, 0)


def _edge_pass(table, edata, zeros_blk):
    mesh = plsc.VectorSubcoreMesh(core_axis_name="c", subcore_axis_name="s")
    f = pl.kernel(
        _edge_pass_body,
        mesh=mesh,
        compiler_params=pltpu.CompilerParams(needs_layout_passes=False),
        out_type=jax.ShapeDtypeStruct((NUM_REL, NPAD, DP), jnp.float32),
        scratch_types=[
            pltpu.VMEM((3, EB), jnp.int32),
            pltpu.VMEM((3, EB), jnp.int32),
            pltpu.VMEM((LROWS, K), jnp.int32),
            pltpu.VMEM((LROWS, K), jnp.int32),
            pltpu.VMEM((K, DP), jnp.float32),
            pltpu.VMEM((K, DP), jnp.float32),
            pltpu.VMEM_SHARED((ACC_ROWS, DP), jnp.float32),
            pltpu.SemaphoreType.DMA,
            pltpu.SemaphoreType.DMA,
        ],
    )
    return f(table, edata, zeros_blk)


def _pad2(w, rows, cols=DP):
    out = jnp.zeros((rows, cols), jnp.float32)
    return out.at[:w.shape[0], :w.shape[1]].set(w)


def _pad1(b, cols=DP):
    return jnp.zeros((cols,), jnp.float32).at[:b.shape[0]].set(b)


def kernel(num_prop, cat_prop, tweet_emb, user_emb, user_name_emb,
           edge_index, edge_type, params):
    p = params
    f32 = jnp.float32
    edata = jnp.concatenate([edge_index.astype(jnp.int32),
                             edge_type.astype(jnp.int32)[None, :]], axis=0)
    edata = edata.reshape(3, 16, EPT)
    pad = jnp.zeros((3, 16, EPT_PAD - EPT), jnp.int32).at[1].set(1 << 20)
    edata = jnp.concatenate([edata, pad], axis=2).transpose(1, 0, 2)

    IND = D // 5
    wn = _pad2(p['W_num'], 5)
    wc = _pad2(p['W_cat'], 3)
    wt = _pad2(p['W_tweet'], 768)
    wu = _pad2(p['W_user'], 768)
    wun = _pad2(p['W_uname'], 768)
    pb = jnp.stack([_pad1(p['b_num']), _pad1(p['b_cat']), _pad1(p['b_tweet']),
                    _pad1(p['b_user']), _pad1(p['b_uname'])])
    win_parts = jnp.stack([
        _pad2(p['W_in'][i * IND:(i + 1) * IND, :], DP) for i in range(5)])
    ones1 = jnp.zeros((3, DP), f32)
    ones1 = ones1.at[0, ONES_LO:].set(1.0)
    ones1 = ones1.at[1, :D].set(p['b_in'])

    root1 = _pad2(p['rgcn1_root'], DP)
    w1 = jnp.stack([_pad2(p['rgcn1_W'][r], DP) for r in range(NUM_REL)])
    b1row = _pad1(p['rgcn1_b'])[None, :]
    root2 = _pad2(p['rgcn2_root'], DP)
    w2 = jnp.stack([_pad2(p['rgcn2_W'][r], DP) for r in range(NUM_REL)])
    b2row = _pad1(p['rgcn2_b'])[None, :]

    BLK = 1000
    grid = (N // BLK,)

    def rowspec(d2=DP):
        return pl.BlockSpec((BLK, d2), lambda i: (i, 0))

    def fullspec(shape):
        nd = len(shape)
        return pl.BlockSpec(shape, lambda i: (0,) * nd)

    x0 = pl.pallas_call(
        _enc_kernel,
        grid=grid,
        in_specs=[
            pl.BlockSpec((BLK, 5), lambda i: (i, 0)),
            pl.BlockSpec((BLK, 3), lambda i: (i, 0)),
            pl.BlockSpec((BLK, 768), lambda i: (i, 0)),
            pl.BlockSpec((BLK, 768), lambda i: (i, 0)),
            pl.BlockSpec((BLK, 768), lambda i: (i, 0)),
            fullspec((5, DP)), fullspec((3, DP)), fullspec((768, DP)),
            fullspec((768, DP)), fullspec((768, DP)), fullspec((5, DP)),
            fullspec((5, DP, DP)), fullspec((3, DP)),
        ],
        out_specs=rowspec(),
        out_shape=jax.ShapeDtypeStruct((N, DP), f32),
    )(num_prop, cat_prop, tweet_emb, user_emb, user_name_emb,
      wn, wc, wt, wu, wun, pb, win_parts, ones1)

    zeros_blk = jnp.zeros((K, DP), f32)

    def combine(x_arr, agg, root, wrel, brow):
        return pl.pallas_call(
            _combine_kernel,
            grid=grid,
            in_specs=[rowspec(),
                      pl.BlockSpec((NUM_REL, BLK, DP), lambda i: (0, i, 0)),
                      fullspec((DP, DP)), fullspec((NUM_REL, DP, DP)),
                      fullspec((1, DP))],
            out_specs=[rowspec(), fullspec((2, DP))],
            out_shape=[jax.ShapeDtypeStruct((N, DP), f32),
                       jax.ShapeDtypeStruct((2, DP), f32)],
            scratch_shapes=[pltpu.VMEM((2, DP), f32)],
        )(x_arr, agg, root, wrel, brow)

    def bn_apply(h, stats, gb):
        return pl.pallas_call(
            _bn_kernel,
            grid=grid,
            in_specs=[rowspec(), fullspec((2, DP)), fullspec((2, DP)),
                      fullspec((3, DP))],
            out_specs=rowspec(),
            out_shape=jax.ShapeDtypeStruct((N, DP), f32),
        )(h, stats, gb, ones1)

    agg1 = _edge_pass(x0, edata, zeros_blk)
    h1, stats1 = combine(x0, agg1, root1, w1, b1row)
    gb1 = jnp.stack([_pad1(p['bn1_g']), _pad1(p['bn1_b'])])
    x1 = bn_apply(h1, stats1, gb1)

    agg2 = _edge_pass(x1, edata, zeros_blk)
    h2, stats2 = combine(x1, agg2, root2, w2, b2row)
    gb2 = jnp.stack([_pad1(p['bn2_g']), _pad1(p['bn2_b'])])

    out = pl.pallas_call(
        _head_kernel,
        grid=grid,
        in_specs=[rowspec(), fullspec((2, DP)), fullspec((2, DP)),
                  fullspec((DP, DP)), fullspec((DP,)),
                  fullspec((DP, DP)), fullspec((DP,)),
                  fullspec((DP, DP)), fullspec((DP,))],
        out_specs=rowspec(),
        out_shape=jax.ShapeDtypeStruct((N, DP), f32),
    )(h2, stats2, gb2, _pad2(p['W_o1'], DP), _pad1(p['b_o1']),
      _pad2(p['W_o2'], DP), _pad1(p['b_o2']), _pad2(p['W_o3'], DP),
      _pad1(p['b_o3']))
    return out[:, :2]


# speedup vs baseline: 7.1887x; 1.3980x over previous
import jax
import jax.numpy as jnp
from jax import lax
from jax.experimental import pallas as pl
from jax.experimental.pallas import tpu as pltpu
from jax.experimental.pallas import tpu_sc as plsc

N = 50000
E = 800000
D = 100
NUM_REL = 3
DP = 128
ONES_LO = 112

CDST = 2560
NCHUNK = 20
NPAD = CDST * NCHUNK
ACC_ROWS = NUM_REL * CDST + 16
EPT = E // 16
EPT_PAD = 51200
EB = 2048
LCAP = 8192
K = 128
LROWS = LCAP // K

HI = lax.Precision.HIGHEST


def _leaky(x):
    return jnp.where(x > 0, x, 0.01 * x)


def _enc_kernel(num_ref, cat_ref, tw_ref, us_ref, un_ref,
                wn_ref, wc_ref, wt_ref, wu_ref, wun_ref, pb_ref,
                win_ref, ones_ref, x_ref):
    f32 = jnp.float32
    parts = []
    for j, (e_ref, w_ref) in enumerate(((num_ref, wn_ref), (cat_ref, wc_ref),
                                        (tw_ref, wt_ref), (us_ref, wu_ref),
                                        (un_ref, wun_ref))):
        part = _leaky(jnp.dot(e_ref[...], w_ref[...],
                              preferred_element_type=f32) + pb_ref[j])
        parts.append(part[:, :D // 5])
    parts.append(jnp.zeros((parts[0].shape[0], DP - D), f32))
    xcat = jnp.concatenate(parts, axis=1)
    x0 = _leaky(jnp.dot(xcat, win_ref[...], preferred_element_type=f32)
                + ones_ref[1][None, :])
    ones = ones_ref[0][None, :]
    x_ref[...] = x0 * (1.0 - ones) + ones


def _combine_kernel(x_ref, agg_ref, root_ref, wrel_ref, bias_ref,
                    h_ref, stats_ref, acc_ref):
    f32 = jnp.float32
    i = pl.program_id(0)
    h = jnp.dot(x_ref[...], root_ref[...], preferred_element_type=f32) \
        + bias_ref[0]
    for r in range(NUM_REL):
        blk = agg_ref[r]
        cnt = jnp.max(blk[:, ONES_LO:], axis=1, keepdims=True)
        mean = blk / jnp.maximum(cnt, 1.0)
        h = h + jnp.dot(mean, wrel_ref[r], preferred_element_type=f32)
    h_ref[...] = h

    @pl.when(i == 0)
    def _():
        acc_ref[...] = jnp.zeros_like(acc_ref)

    acc_ref[0, :] += jnp.sum(h, axis=0)
    acc_ref[1, :] += jnp.sum(h * h, axis=0)

    @pl.when(i == pl.num_programs(0) - 1)
    def _():
        stats_ref[...] = acc_ref[...]


def _bn(h, stats_ref, gb_ref):
    mean = stats_ref[0] * (1.0 / N)
    var = stats_ref[1] * (1.0 / N) - mean * mean
    scale = gb_ref[0] / jnp.sqrt(var + 1e-5)
    return _leaky((h - mean) * scale + gb_ref[1])


def _bn_kernel(h_ref, stats_ref, gb_ref, ones_ref, x_ref):
    x = _bn(h_ref[...], stats_ref, gb_ref)
    ones = ones_ref[0][None, :]
    x_ref[...] = x * (1.0 - ones) + ones


def _head_kernel(h_ref, stats_ref, gb_ref, w1_ref, b1_ref, w2_ref, b2_ref,
                 w3_ref, b3_ref, o_ref):
    f32 = jnp.float32
    x = _bn(h_ref[...], stats_ref, gb_ref)
    x = _leaky(jnp.dot(x, w1_ref[...], preferred_element_type=f32) + b1_ref[...])
    x = _leaky(jnp.dot(x, w2_ref[...], preferred_element_type=f32) + b2_ref[...])
    o_ref[...] = jnp.dot(x, w3_ref[...], preferred_element_type=f32) + b3_ref[...]


def _edge_pass_body(tab_hbm, edata_hbm, zeros_hbm,
                    agg_hbm,
                    e_a, e_b, glist, alist, buf, zbuf,
                    acc, sem, sem_e):
    cid = lax.axis_index("c")
    sid = lax.axis_index("s")
    e0 = sid * EPT
    trash = ACC_ROWS - 16 + sid
    lanes = lax.iota(jnp.int32, 16)
    NB = EPT_PAD // EB

    pltpu.sync_copy(zeros_hbm, zbuf)

    def stage(b, dst_ref):
        s0 = pl.multiple_of(b * EB, EB)
        return pltpu.async_copy(edata_hbm.at[sid, :, pl.ds(s0, EB)], dst_ref,
                                sem_e)

    def flush(j, done):
        row = (done // K) & (LROWS - 1)
        pltpu.async_copy(tab_hbm.at[glist.at[row]], buf, sem).wait()
        pltpu.sync_copy(buf, acc.at[alist.at[row]], add=True)
        return done + K

    def chunk_body(k, _unused):
        base_dst = (cid * (NCHUNK // 2) + k) * CDST
        z0 = sid * (ACC_ROWS // 16)
        for off in range(0, ACC_ROWS // 16, K):
            ln = min(K, ACC_ROWS // 16 - off)
            pltpu.sync_copy(zbuf.at[pl.ds(0, ln)], acc.at[pl.ds(z0 + off, ln)])
        plsc.subcore_barrier()

        def scan(e_ref, carry):
            off_s, done = carry

            def vit(i, offv):
                ii = pl.multiple_of(i * 16, 16)
                s = e_ref[0, pl.ds(ii, 16)]
                d = e_ref[1, pl.ds(ii, 16)]
                r = e_ref[2, pl.ds(ii, 16)]
                dl = d - base_dst
                m = (dl >= 0) & (dl < CDST)
                ai = r * CDST + dl
                pos = offv + plsc.cumsum(m.astype(jnp.int32)) - 1
                prow = (pos // K) & (LROWS - 1)
                pcol = pos & (K - 1)
                plsc.store_scatter(glist, [prow, pcol], s, mask=m)
                plsc.store_scatter(alist, [prow, pcol], ai, mask=m)
                return offv + plsc.all_reduce_population_count(m)

            offv = lax.fori_loop(0, EB // 16, vit,
                                 jnp.full((16,), off_s, jnp.int32))
            off_s2 = jnp.max(offv)
            nb = (off_s2 - done) // K
            done2 = lax.fori_loop(0, nb, flush, done)
            return off_s2, done2

        stage(0, e_a).wait()

        def pair(j, carry):
            cp_b = stage(2 * j + 1, e_b)
            carry = scan(e_a, carry)
            cp_b.wait()
            cp_a = stage(2 * j + 2, e_a)
            carry = scan(e_b, carry)
            cp_a.wait()
            return carry

        carry = lax.fori_loop(0, (NB - 1) // 2, pair,
                              (jnp.int32(0), jnp.int32(0)))
        off_s, done = scan(e_a, carry)
        rem = off_s - done

        def padfill(i, _):
            pos = off_s + i * 16 + lanes
            m = pos < done + K
            prow = (pos // K) & (LROWS - 1)
            pcol = pos & (K - 1)
            plsc.store_scatter(glist, [prow, pcol],
                               jnp.full((16,), sid * 64, jnp.int32), mask=m)
            plsc.store_scatter(alist, [prow, pcol],
                               jnp.full((16,), trash, jnp.int32), mask=m)
            return 0

        @pl.when(rem > 0)
        def _():
            lax.fori_loop(0, K // 16, padfill, 0)
            flush(0, done)

        plsc.subcore_barrier()
        for r in range(NUM_REL):
            pltpu.sync_copy(
                acc.at[pl.ds(r * CDST + sid * (CDST // 16), CDST // 16)],
                agg_hbm.at[r, pl.ds(base_dst + sid * (CDST // 16), CDST // 16)])
        plsc.subcore_barrier()
        return 0

    lax.fori_loop(0, NCHUNK // 2, chunk_body, 0)


def _edge_pass(table, edata, zeros_blk):
    mesh = plsc.VectorSubcoreMesh(core_axis_name="c", subcore_axis_name="s")
    f = pl.kernel(
        _edge_pass_body,
        mesh=mesh,
        compiler_params=pltpu.CompilerParams(needs_layout_passes=False),
        out_type=jax.ShapeDtypeStruct((NUM_REL, NPAD, DP), jnp.float32),
        scratch_types=[
            pltpu.VMEM((3, EB), jnp.int32),
            pltpu.VMEM((3, EB), jnp.int32),
            pltpu.VMEM((LROWS, K), jnp.int32),
            pltpu.VMEM((LROWS, K), jnp.int32),
            pltpu.VMEM((K, DP), jnp.float32),
            pltpu.VMEM((K, DP), jnp.float32),
            pltpu.VMEM_SHARED((ACC_ROWS, DP), jnp.float32),
            pltpu.SemaphoreType.DMA,
            pltpu.SemaphoreType.DMA,
        ],
    )
    return f(table, edata, zeros_blk)


def _pad2(w, rows, cols=DP):
    out = jnp.zeros((rows, cols), jnp.float32)
    return out.at[:w.shape[0], :w.shape[1]].set(w)


def _pad1(b, cols=DP):
    return jnp.zeros((cols,), jnp.float32).at[:b.shape[0]].set(b)


def kernel(num_prop, cat_prop, tweet_emb, user_emb, user_name_emb,
           edge_index, edge_type, params):
    p = params
    f32 = jnp.float32
    edata = jnp.concatenate([edge_index.astype(jnp.int32),
                             edge_type.astype(jnp.int32)[None, :]], axis=0)
    edata = edata.reshape(3, 16, EPT)
    pad = jnp.zeros((3, 16, EPT_PAD - EPT), jnp.int32).at[1].set(1 << 20)
    edata = jnp.concatenate([edata, pad], axis=2).transpose(1, 0, 2)

    IND = D // 5
    wn = _pad2(p['W_num'], 5)
    wc = _pad2(p['W_cat'], 3)
    wt = _pad2(p['W_tweet'], 768)
    wu = _pad2(p['W_user'], 768)
    wun = _pad2(p['W_uname'], 768)
    pb = jnp.stack([_pad1(p['b_num']), _pad1(p['b_cat']), _pad1(p['b_tweet']),
                    _pad1(p['b_user']), _pad1(p['b_uname'])])
    win = _pad2(p['W_in'], DP)
    ones1 = jnp.zeros((3, DP), f32)
    ones1 = ones1.at[0, ONES_LO:].set(1.0)
    ones1 = ones1.at[1, :D].set(p['b_in'])

    root1 = _pad2(p['rgcn1_root'], DP)
    w1 = jnp.stack([_pad2(p['rgcn1_W'][r], DP) for r in range(NUM_REL)])
    b1row = _pad1(p['rgcn1_b'])[None, :]
    root2 = _pad2(p['rgcn2_root'], DP)
    w2 = jnp.stack([_pad2(p['rgcn2_W'][r], DP) for r in range(NUM_REL)])
    b2row = _pad1(p['rgcn2_b'])[None, :]

    BLK = 1000
    grid = (N // BLK,)

    def rowspec(d2=DP):
        return pl.BlockSpec((BLK, d2), lambda i: (i, 0))

    def fullspec(shape):
        nd = len(shape)
        return pl.BlockSpec(shape, lambda i: (0,) * nd)

    x0 = pl.pallas_call(
        _enc_kernel,
        grid=grid,
        in_specs=[
            pl.BlockSpec((BLK, 5), lambda i: (i, 0)),
            pl.BlockSpec((BLK, 3), lambda i: (i, 0)),
            pl.BlockSpec((BLK, 768), lambda i: (i, 0)),
            pl.BlockSpec((BLK, 768), lambda i: (i, 0)),
            pl.BlockSpec((BLK, 768), lambda i: (i, 0)),
            fullspec((5, DP)), fullspec((3, DP)), fullspec((768, DP)),
            fullspec((768, DP)), fullspec((768, DP)), fullspec((5, DP)),
            fullspec((DP, DP)), fullspec((3, DP)),
        ],
        out_specs=rowspec(),
        out_shape=jax.ShapeDtypeStruct((N, DP), f32),
    )(num_prop, cat_prop, tweet_emb, user_emb, user_name_emb,
      wn, wc, wt, wu, wun, pb, win, ones1)

    zeros_blk = jnp.zeros((K, DP), f32)

    def combine(x_arr, agg, root, wrel, brow):
        return pl.pallas_call(
            _combine_kernel,
            grid=grid,
            in_specs=[rowspec(),
                      pl.BlockSpec((NUM_REL, BLK, DP), lambda i: (0, i, 0)),
                      fullspec((DP, DP)), fullspec((NUM_REL, DP, DP)),
                      fullspec((1, DP))],
            out_specs=[rowspec(), fullspec((2, DP))],
            out_shape=[jax.ShapeDtypeStruct((N, DP), f32),
                       jax.ShapeDtypeStruct((2, DP), f32)],
            scratch_shapes=[pltpu.VMEM((2, DP), f32)],
        )(x_arr, agg, root, wrel, brow)

    def bn_apply(h, stats, gb):
        return pl.pallas_call(
            _bn_kernel,
            grid=grid,
            in_specs=[rowspec(), fullspec((2, DP)), fullspec((2, DP)),
                      fullspec((3, DP))],
            out_specs=rowspec(),
            out_shape=jax.ShapeDtypeStruct((N, DP), f32),
        )(h, stats, gb, ones1)

    agg1 = _edge_pass(x0, edata, zeros_blk)
    h1, stats1 = combine(x0, agg1, root1, w1, b1row)
    gb1 = jnp.stack([_pad1(p['bn1_g']), _pad1(p['bn1_b'])])
    x1 = bn_apply(h1, stats1, gb1)

    agg2 = _edge_pass(x1, edata, zeros_blk)
    h2, stats2 = combine(x1, agg2, root2, w2, b2row)
    gb2 = jnp.stack([_pad1(p['bn2_g']), _pad1(p['bn2_b'])])

    out = pl.pallas_call(
        _head_kernel,
        grid=grid,
        in_specs=[rowspec(), fullspec((2, DP)), fullspec((2, DP)),
                  fullspec((DP, DP)), fullspec((DP,)),
                  fullspec((DP, DP)), fullspec((DP,)),
                  fullspec((DP, DP)), fullspec((DP,))],
        out_specs=rowspec(),
        out_shape=jax.ShapeDtypeStruct((N, DP), f32),
    )(h2, stats2, gb2, _pad2(p['W_o1'], DP), _pad1(p['b_o1']),
      _pad2(p['W_o2'], DP), _pad1(p['b_o2']), _pad2(p['W_o3'], DP),
      _pad1(p['b_o3']))
    return out[:, :2]


# speedup vs baseline: 7.2581x; 1.0097x over previous
import jax
import jax.numpy as jnp
from jax import lax
from jax.experimental import pallas as pl
from jax.experimental.pallas import tpu as pltpu
from jax.experimental.pallas import tpu_sc as plsc

N = 50000
E = 800000
D = 100
NUM_REL = 3
DP = 128
ONES_LO = 112

CDST = 2560
NCHUNK = 20
NPAD = CDST * NCHUNK
ACC_ROWS = NUM_REL * CDST + 16
EPT = E // 16
EPT_PAD = 51200
EB = 2048
LCAP = 8192
K = 128
LROWS = LCAP // K

HI = lax.Precision.HIGHEST


def _leaky(x):
    return jnp.where(x > 0, x, 0.01 * x)


def _enc_kernel(num_ref, cat_ref, tw_ref, us_ref, un_ref,
                wn_ref, wc_ref, wt_ref, wu_ref, wun_ref, pb_ref,
                win_ref, ones_ref, x_ref):
    f32 = jnp.float32
    parts = []
    for j, (e_ref, w_ref) in enumerate(((num_ref, wn_ref), (cat_ref, wc_ref),
                                        (tw_ref, wt_ref), (us_ref, wu_ref),
                                        (un_ref, wun_ref))):
        part = _leaky(jnp.dot(e_ref[...], w_ref[...],
                              preferred_element_type=f32) + pb_ref[j])
        parts.append(part[:, :D // 5])
    parts.append(jnp.zeros((parts[0].shape[0], DP - D), f32))
    xcat = jnp.concatenate(parts, axis=1)
    x0 = _leaky(jnp.dot(xcat, win_ref[...], preferred_element_type=f32)
                + ones_ref[1][None, :])
    ones = ones_ref[0][None, :]
    x_ref[...] = x0 * (1.0 - ones) + ones


def _combine_kernel(x_ref, agg_ref, root_ref, wrel_ref, bias_ref,
                    h_ref, stats_ref, acc_ref):
    f32 = jnp.float32
    i = pl.program_id(0)
    h = jnp.dot(x_ref[...], root_ref[...], preferred_element_type=f32) \
        + bias_ref[0]
    for r in range(NUM_REL):
        blk = agg_ref[r]
        cnt = jnp.max(blk[:, ONES_LO:], axis=1, keepdims=True)
        mean = blk / jnp.maximum(cnt, 1.0)
        h = h + jnp.dot(mean, wrel_ref[r], preferred_element_type=f32)
    h_ref[...] = h

    @pl.when(i == 0)
    def _():
        acc_ref[...] = jnp.zeros_like(acc_ref)

    acc_ref[0, :] += jnp.sum(h, axis=0)
    acc_ref[1, :] += jnp.sum(h * h, axis=0)

    @pl.when(i == pl.num_programs(0) - 1)
    def _():
        stats_ref[...] = acc_ref[...]


def _bn(h, stats_ref, gb_ref):
    mean = stats_ref[0] * (1.0 / N)
    var = stats_ref[1] * (1.0 / N) - mean * mean
    scale = gb_ref[0] / jnp.sqrt(var + 1e-5)
    return _leaky((h - mean) * scale + gb_ref[1])


def _bn_kernel(h_ref, stats_ref, gb_ref, ones_ref, x_ref):
    x = _bn(h_ref[...], stats_ref, gb_ref)
    ones = ones_ref[0][None, :]
    x_ref[...] = x * (1.0 - ones) + ones


def _head_kernel(h_ref, stats_ref, gb_ref, w1_ref, b1_ref, w2_ref, b2_ref,
                 w3_ref, b3_ref, o_ref):
    f32 = jnp.float32
    x = _bn(h_ref[...], stats_ref, gb_ref)
    x = _leaky(jnp.dot(x, w1_ref[...], preferred_element_type=f32) + b1_ref[...])
    x = _leaky(jnp.dot(x, w2_ref[...], preferred_element_type=f32) + b2_ref[...])
    o_ref[...] = jnp.dot(x, w3_ref[...], preferred_element_type=f32) + b3_ref[...]


def _edge_pass_body(tab_hbm, edata_hbm, zeros_hbm,
                    agg_hbm,
                    e_a, e_b, glist, alist, buf, zbuf,
                    acc, sem, sem_e):
    cid = lax.axis_index("c")
    sid = lax.axis_index("s")
    e0 = sid * EPT
    trash = ACC_ROWS - 16 + sid
    lanes = lax.iota(jnp.int32, 16)
    lane15 = jnp.full((16,), 15, jnp.int32)
    NB = EPT_PAD // EB

    pltpu.sync_copy(zeros_hbm, zbuf)

    def stage(b, dst_ref):
        s0 = pl.multiple_of(b * EB, EB)
        return pltpu.async_copy(edata_hbm.at[sid, :, pl.ds(s0, EB)], dst_ref,
                                sem_e)

    def flush(j, done):
        row = (done // K) & (LROWS - 1)
        pltpu.async_copy(tab_hbm.at[glist.at[row]], buf, sem).wait()
        pltpu.sync_copy(buf, acc.at[alist.at[row]], add=True)
        return done + K

    def chunk_body(k, _unused):
        base_dst = (cid * (NCHUNK // 2) + k) * CDST
        z0 = sid * (ACC_ROWS // 16)
        for off in range(0, ACC_ROWS // 16, K):
            ln = min(K, ACC_ROWS // 16 - off)
            pltpu.sync_copy(zbuf.at[pl.ds(0, ln)], acc.at[pl.ds(z0 + off, ln)])
        plsc.subcore_barrier()

        def scan(e_ref, carry):
            off_s, done = carry

            def vit(i, offv):
                ii = pl.multiple_of(i * 16, 16)
                s = e_ref[0, pl.ds(ii, 16)]
                d = e_ref[1, pl.ds(ii, 16)]
                r = e_ref[2, pl.ds(ii, 16)]
                dl = d - base_dst
                m = (dl >= 0) & (dl < CDST)
                ai = r * CDST + dl
                pos = offv + plsc.cumsum(m.astype(jnp.int32)) - 1
                prow = (pos // K) & (LROWS - 1)
                pcol = pos & (K - 1)
                plsc.store_scatter(glist, [prow, pcol], s, mask=m)
                plsc.store_scatter(alist, [prow, pcol], ai, mask=m)
                return pos.at[lane15].get(mode="promise_in_bounds") + 1

            offv = lax.fori_loop(0, EB // 16, vit,
                                 jnp.full((16,), off_s, jnp.int32))
            off_s2 = jnp.max(offv)
            nb = (off_s2 - done) // K
            done2 = lax.fori_loop(0, nb, flush, done)
            return off_s2, done2

        stage(0, e_a).wait()

        def pair(j, carry):
            cp_b = stage(2 * j + 1, e_b)
            carry = scan(e_a, carry)
            cp_b.wait()
            cp_a = stage(2 * j + 2, e_a)
            carry = scan(e_b, carry)
            cp_a.wait()
            return carry

        carry = lax.fori_loop(0, (NB - 1) // 2, pair,
                              (jnp.int32(0), jnp.int32(0)))
        off_s, done = scan(e_a, carry)
        rem = off_s - done

        def padfill(i, _):
            pos = off_s + i * 16 + lanes
            m = pos < done + K
            prow = (pos // K) & (LROWS - 1)
            pcol = pos & (K - 1)
            plsc.store_scatter(glist, [prow, pcol],
                               jnp.full((16,), sid * 64, jnp.int32), mask=m)
            plsc.store_scatter(alist, [prow, pcol],
                               jnp.full((16,), trash, jnp.int32), mask=m)
            return 0

        @pl.when(rem > 0)
        def _():
            lax.fori_loop(0, K // 16, padfill, 0)
            flush(0, done)

        plsc.subcore_barrier()
        for r in range(NUM_REL):
            pltpu.sync_copy(
                acc.at[pl.ds(r * CDST + sid * (CDST // 16), CDST // 16)],
                agg_hbm.at[r, pl.ds(base_dst + sid * (CDST // 16), CDST // 16)])
        plsc.subcore_barrier()
        return 0

    lax.fori_loop(0, NCHUNK // 2, chunk_body, 0)


def _edge_pass(table, edata, zeros_blk):
    mesh = plsc.VectorSubcoreMesh(core_axis_name="c", subcore_axis_name="s")
    f = pl.kernel(
        _edge_pass_body,
        mesh=mesh,
        compiler_params=pltpu.CompilerParams(needs_layout_passes=False),
        out_type=jax.ShapeDtypeStruct((NUM_REL, NPAD, DP), jnp.float32),
        scratch_types=[
            pltpu.VMEM((3, EB), jnp.int32),
            pltpu.VMEM((3, EB), jnp.int32),
            pltpu.VMEM((LROWS, K), jnp.int32),
            pltpu.VMEM((LROWS, K), jnp.int32),
            pltpu.VMEM((K, DP), jnp.float32),
            pltpu.VMEM((K, DP), jnp.float32),
            pltpu.VMEM_SHARED((ACC_ROWS, DP), jnp.float32),
            pltpu.SemaphoreType.DMA,
            pltpu.SemaphoreType.DMA,
        ],
    )
    return f(table, edata, zeros_blk)


def _pad2(w, rows, cols=DP):
    out = jnp.zeros((rows, cols), jnp.float32)
    return out.at[:w.shape[0], :w.shape[1]].set(w)


def _pad1(b, cols=DP):
    return jnp.zeros((cols,), jnp.float32).at[:b.shape[0]].set(b)


def kernel(num_prop, cat_prop, tweet_emb, user_emb, user_name_emb,
           edge_index, edge_type, params):
    p = params
    f32 = jnp.float32
    edata = jnp.concatenate([edge_index.astype(jnp.int32),
                             edge_type.astype(jnp.int32)[None, :]], axis=0)
    edata = edata.reshape(3, 16, EPT)
    pad = jnp.zeros((3, 16, EPT_PAD - EPT), jnp.int32).at[1].set(1 << 20)
    edata = jnp.concatenate([edata, pad], axis=2).transpose(1, 0, 2)

    IND = D // 5
    wn = _pad2(p['W_num'], 5)
    wc = _pad2(p['W_cat'], 3)
    wt = _pad2(p['W_tweet'], 768)
    wu = _pad2(p['W_user'], 768)
    wun = _pad2(p['W_uname'], 768)
    pb = jnp.stack([_pad1(p['b_num']), _pad1(p['b_cat']), _pad1(p['b_tweet']),
                    _pad1(p['b_user']), _pad1(p['b_uname'])])
    win = _pad2(p['W_in'], DP)
    ones1 = jnp.zeros((3, DP), f32)
    ones1 = ones1.at[0, ONES_LO:].set(1.0)
    ones1 = ones1.at[1, :D].set(p['b_in'])

    root1 = _pad2(p['rgcn1_root'], DP)
    w1 = jnp.stack([_pad2(p['rgcn1_W'][r], DP) for r in range(NUM_REL)])
    b1row = _pad1(p['rgcn1_b'])[None, :]
    root2 = _pad2(p['rgcn2_root'], DP)
    w2 = jnp.stack([_pad2(p['rgcn2_W'][r], DP) for r in range(NUM_REL)])
    b2row = _pad1(p['rgcn2_b'])[None, :]

    BLK = 1000
    grid = (N // BLK,)

    def rowspec(d2=DP):
        return pl.BlockSpec((BLK, d2), lambda i: (i, 0))

    def fullspec(shape):
        nd = len(shape)
        return pl.BlockSpec(shape, lambda i: (0,) * nd)

    x0 = pl.pallas_call(
        _enc_kernel,
        grid=grid,
        in_specs=[
            pl.BlockSpec((BLK, 5), lambda i: (i, 0)),
            pl.BlockSpec((BLK, 3), lambda i: (i, 0)),
            pl.BlockSpec((BLK, 768), lambda i: (i, 0)),
            pl.BlockSpec((BLK, 768), lambda i: (i, 0)),
            pl.BlockSpec((BLK, 768), lambda i: (i, 0)),
            fullspec((5, DP)), fullspec((3, DP)), fullspec((768, DP)),
            fullspec((768, DP)), fullspec((768, DP)), fullspec((5, DP)),
            fullspec((DP, DP)), fullspec((3, DP)),
        ],
        out_specs=rowspec(),
        out_shape=jax.ShapeDtypeStruct((N, DP), f32),
    )(num_prop, cat_prop, tweet_emb, user_emb, user_name_emb,
      wn, wc, wt, wu, wun, pb, win, ones1)

    zeros_blk = jnp.zeros((K, DP), f32)

    def combine(x_arr, agg, root, wrel, brow):
        return pl.pallas_call(
            _combine_kernel,
            grid=grid,
            in_specs=[rowspec(),
                      pl.BlockSpec((NUM_REL, BLK, DP), lambda i: (0, i, 0)),
                      fullspec((DP, DP)), fullspec((NUM_REL, DP, DP)),
                      fullspec((1, DP))],
            out_specs=[rowspec(), fullspec((2, DP))],
            out_shape=[jax.ShapeDtypeStruct((N, DP), f32),
                       jax.ShapeDtypeStruct((2, DP), f32)],
            scratch_shapes=[pltpu.VMEM((2, DP), f32)],
        )(x_arr, agg, root, wrel, brow)

    def bn_apply(h, stats, gb):
        return pl.pallas_call(
            _bn_kernel,
            grid=grid,
            in_specs=[rowspec(), fullspec((2, DP)), fullspec((2, DP)),
                      fullspec((3, DP))],
            out_specs=rowspec(),
            out_shape=jax.ShapeDtypeStruct((N, DP), f32),
        )(h, stats, gb, ones1)

    agg1 = _edge_pass(x0, edata, zeros_blk)
    h1, stats1 = combine(x0, agg1, root1, w1, b1row)
    gb1 = jnp.stack([_pad1(p['bn1_g']), _pad1(p['bn1_b'])])
    x1 = bn_apply(h1, stats1, gb1)

    agg2 = _edge_pass(x1, edata, zeros_blk)
    h2, stats2 = combine(x1, agg2, root2, w2, b2row)
    gb2 = jnp.stack([_pad1(p['bn2_g']), _pad1(p['bn2_b'])])

    out = pl.pallas_call(
        _head_kernel,
        grid=grid,
        in_specs=[rowspec(), fullspec((2, DP)), fullspec((2, DP)),
                  fullspec((DP, DP)), fullspec((DP,)),
                  fullspec((DP, DP)), fullspec((DP,)),
                  fullspec((DP, DP)), fullspec((DP,))],
        out_specs=rowspec(),
        out_shape=jax.ShapeDtypeStruct((N, DP), f32),
    )(h2, stats2, gb2, _pad2(p['W_o1'], DP), _pad1(p['b_o1']),
      _pad2(p['W_o2'], DP), _pad1(p['b_o2']), _pad2(p['W_o3'], DP),
      _pad1(p['b_o3']))
    return out[:, :2]


# speedup vs baseline: 9.9888x; 1.3762x over previous
import jax
import jax.numpy as jnp
from jax import lax
from jax.experimental import pallas as pl
from jax.experimental.pallas import tpu as pltpu
from jax.experimental.pallas import tpu_sc as plsc

N = 50000
E = 800000
D = 100
NUM_REL = 3
DP = 128
ONES_LO = 112

CDST = 2560
NCHUNK = 20
NPAD = CDST * NCHUNK
ACC_ROWS = NUM_REL * CDST + 16
EPT = E // 16
EPT_PAD = 51200
EB = 2048
LCAP = 8192
K = 128
LROWS = LCAP // K

HI = lax.Precision.HIGHEST


def _leaky(x):
    return jnp.where(x > 0, x, 0.01 * x)


def _enc_kernel(num_ref, cat_ref, tw_ref, us_ref, un_ref,
                wn_ref, wc_ref, wt_ref, wu_ref, wun_ref, pb_ref,
                win_ref, ones_ref, x_ref):
    f32 = jnp.float32
    parts = []
    for j, (e_ref, w_ref) in enumerate(((num_ref, wn_ref), (cat_ref, wc_ref),
                                        (tw_ref, wt_ref), (us_ref, wu_ref),
                                        (un_ref, wun_ref))):
        part = _leaky(jnp.dot(e_ref[...], w_ref[...],
                              preferred_element_type=f32) + pb_ref[j])
        parts.append(part[:, :D // 5])
    parts.append(jnp.zeros((parts[0].shape[0], DP - D), f32))
    xcat = jnp.concatenate(parts, axis=1)
    x0 = _leaky(jnp.dot(xcat, win_ref[...], preferred_element_type=f32)
                + ones_ref[1][None, :])
    ones = ones_ref[0][None, :]
    x_ref[...] = x0 * (1.0 - ones) + ones


def _combine_kernel(x_ref, agg_ref, root_ref, wrel_ref, bias_ref,
                    h_ref, stats_ref, acc_ref):
    f32 = jnp.float32
    i = pl.program_id(0)
    h = jnp.dot(x_ref[...], root_ref[...], preferred_element_type=f32) \
        + bias_ref[0]
    for r in range(NUM_REL):
        blk = agg_ref[r]
        cnt = jnp.max(blk[:, ONES_LO:], axis=1, keepdims=True)
        mean = blk / jnp.maximum(cnt, 1.0)
        h = h + jnp.dot(mean, wrel_ref[r], preferred_element_type=f32)
    h_ref[...] = h

    @pl.when(i == 0)
    def _():
        acc_ref[...] = jnp.zeros_like(acc_ref)

    acc_ref[0, :] += jnp.sum(h, axis=0)
    acc_ref[1, :] += jnp.sum(h * h, axis=0)

    @pl.when(i == pl.num_programs(0) - 1)
    def _():
        stats_ref[...] = acc_ref[...]


def _bn(h, stats_ref, gb_ref):
    mean = stats_ref[0] * (1.0 / N)
    var = stats_ref[1] * (1.0 / N) - mean * mean
    scale = gb_ref[0] / jnp.sqrt(var + 1e-5)
    return _leaky((h - mean) * scale + gb_ref[1])


def _bn_kernel(h_ref, stats_ref, gb_ref, ones_ref, x_ref):
    x = _bn(h_ref[...], stats_ref, gb_ref)
    ones = ones_ref[0][None, :]
    x_ref[...] = x * (1.0 - ones) + ones


def _head_kernel(h_ref, stats_ref, gb_ref, w1_ref, b1_ref, w2_ref, b2_ref,
                 w3_ref, b3_ref, o_ref):
    f32 = jnp.float32
    x = _bn(h_ref[...], stats_ref, gb_ref)
    x = _leaky(jnp.dot(x, w1_ref[...], preferred_element_type=f32) + b1_ref[...])
    x = _leaky(jnp.dot(x, w2_ref[...], preferred_element_type=f32) + b2_ref[...])
    o_ref[...] = jnp.dot(x, w3_ref[...], preferred_element_type=f32) + b3_ref[...]


def _edge_pass_body(tab_hbm, edata_hbm, zeros_hbm,
                    agg_hbm,
                    e_a, e_b, glist, alist, buf, zbuf,
                    acc, sem, sem_e):
    cid = lax.axis_index("c")
    sid = lax.axis_index("s")
    e0 = sid * EPT
    trash = ACC_ROWS - 16 + sid
    lanes = lax.iota(jnp.int32, 16)
    lane15 = jnp.full((16,), 15, jnp.int32)
    NB = EPT_PAD // EB

    pltpu.sync_copy(zeros_hbm, zbuf)

    def stage(b, dst_ref):
        s0 = pl.multiple_of(b * EB, EB)
        return pltpu.async_copy(edata_hbm.at[sid, :, pl.ds(s0, EB)], dst_ref,
                                sem_e)

    def flush(j, done):
        row = (done // K) & (LROWS - 1)
        pltpu.async_copy(tab_hbm.at[glist.at[row]], buf, sem).wait()
        pltpu.sync_copy(buf, acc.at[alist.at[row]], add=True)
        return done + K

    def chunk_body(k, _unused):
        base_dst = (cid * (NCHUNK // 2) + k) * CDST
        z0 = sid * (ACC_ROWS // 16)
        for off in range(0, ACC_ROWS // 16, K):
            ln = min(K, ACC_ROWS // 16 - off)
            pltpu.sync_copy(zbuf.at[pl.ds(0, ln)], acc.at[pl.ds(z0 + off, ln)])
        plsc.subcore_barrier()

        def scan(e_ref, carry):
            off_s, done = carry

            @plsc.parallel_loop(0, EB // 16, unroll=4,
                                carry=jnp.full((16,), off_s, jnp.int32))
            def vit(i, offv):
                ii = pl.multiple_of(i * 16, 16)
                s = e_ref[0, pl.ds(ii, 16)]
                d = e_ref[1, pl.ds(ii, 16)]
                r = e_ref[2, pl.ds(ii, 16)]
                dl = d - base_dst
                m = (dl >= 0) & (dl < CDST)
                ai = r * CDST + dl
                pos = offv + plsc.cumsum(m.astype(jnp.int32)) - 1
                prow = (pos // K) & (LROWS - 1)
                pcol = pos & (K - 1)
                plsc.store_scatter(glist, [prow, pcol], s, mask=m)
                plsc.store_scatter(alist, [prow, pcol], ai, mask=m)
                return pos.at[lane15].get(mode="promise_in_bounds") + 1

            off_s2 = jnp.max(vit)
            nb = (off_s2 - done) // K
            done2 = lax.fori_loop(0, nb, flush, done)
            return off_s2, done2

        stage(0, e_a).wait()

        def pair(j, carry):
            cp_b = stage(2 * j + 1, e_b)
            carry = scan(e_a, carry)
            cp_b.wait()
            cp_a = stage(2 * j + 2, e_a)
            carry = scan(e_b, carry)
            cp_a.wait()
            return carry

        carry = lax.fori_loop(0, (NB - 1) // 2, pair,
                              (jnp.int32(0), jnp.int32(0)))
        off_s, done = scan(e_a, carry)
        rem = off_s - done

        def padfill(i, _):
            pos = off_s + i * 16 + lanes
            m = pos < done + K
            prow = (pos // K) & (LROWS - 1)
            pcol = pos & (K - 1)
            plsc.store_scatter(glist, [prow, pcol],
                               jnp.full((16,), sid * 64, jnp.int32), mask=m)
            plsc.store_scatter(alist, [prow, pcol],
                               jnp.full((16,), trash, jnp.int32), mask=m)
            return 0

        @pl.when(rem > 0)
        def _():
            lax.fori_loop(0, K // 16, padfill, 0)
            flush(0, done)

        plsc.subcore_barrier()
        for r in range(NUM_REL):
            pltpu.sync_copy(
                acc.at[pl.ds(r * CDST + sid * (CDST // 16), CDST // 16)],
                agg_hbm.at[r, pl.ds(base_dst + sid * (CDST // 16), CDST // 16)])
        plsc.subcore_barrier()
        return 0

    lax.fori_loop(0, NCHUNK // 2, chunk_body, 0)


def _edge_pass(table, edata, zeros_blk):
    mesh = plsc.VectorSubcoreMesh(core_axis_name="c", subcore_axis_name="s")
    f = pl.kernel(
        _edge_pass_body,
        mesh=mesh,
        compiler_params=pltpu.CompilerParams(needs_layout_passes=False),
        out_type=jax.ShapeDtypeStruct((NUM_REL, NPAD, DP), jnp.float32),
        scratch_types=[
            pltpu.VMEM((3, EB), jnp.int32),
            pltpu.VMEM((3, EB), jnp.int32),
            pltpu.VMEM((LROWS, K), jnp.int32),
            pltpu.VMEM((LROWS, K), jnp.int32),
            pltpu.VMEM((K, DP), jnp.float32),
            pltpu.VMEM((K, DP), jnp.float32),
            pltpu.VMEM_SHARED((ACC_ROWS, DP), jnp.float32),
            pltpu.SemaphoreType.DMA,
            pltpu.SemaphoreType.DMA,
        ],
    )
    return f(table, edata, zeros_blk)


def _pad2(w, rows, cols=DP):
    out = jnp.zeros((rows, cols), jnp.float32)
    return out.at[:w.shape[0], :w.shape[1]].set(w)


def _pad1(b, cols=DP):
    return jnp.zeros((cols,), jnp.float32).at[:b.shape[0]].set(b)


def kernel(num_prop, cat_prop, tweet_emb, user_emb, user_name_emb,
           edge_index, edge_type, params):
    p = params
    f32 = jnp.float32
    edata = jnp.concatenate([edge_index.astype(jnp.int32),
                             edge_type.astype(jnp.int32)[None, :]], axis=0)
    edata = edata.reshape(3, 16, EPT)
    pad = jnp.zeros((3, 16, EPT_PAD - EPT), jnp.int32).at[1].set(1 << 20)
    edata = jnp.concatenate([edata, pad], axis=2).transpose(1, 0, 2)

    IND = D // 5
    wn = _pad2(p['W_num'], 5)
    wc = _pad2(p['W_cat'], 3)
    wt = _pad2(p['W_tweet'], 768)
    wu = _pad2(p['W_user'], 768)
    wun = _pad2(p['W_uname'], 768)
    pb = jnp.stack([_pad1(p['b_num']), _pad1(p['b_cat']), _pad1(p['b_tweet']),
                    _pad1(p['b_user']), _pad1(p['b_uname'])])
    win = _pad2(p['W_in'], DP)
    ones1 = jnp.zeros((3, DP), f32)
    ones1 = ones1.at[0, ONES_LO:].set(1.0)
    ones1 = ones1.at[1, :D].set(p['b_in'])

    root1 = _pad2(p['rgcn1_root'], DP)
    w1 = jnp.stack([_pad2(p['rgcn1_W'][r], DP) for r in range(NUM_REL)])
    b1row = _pad1(p['rgcn1_b'])[None, :]
    root2 = _pad2(p['rgcn2_root'], DP)
    w2 = jnp.stack([_pad2(p['rgcn2_W'][r], DP) for r in range(NUM_REL)])
    b2row = _pad1(p['rgcn2_b'])[None, :]

    BLK = 1000
    grid = (N // BLK,)

    def rowspec(d2=DP):
        return pl.BlockSpec((BLK, d2), lambda i: (i, 0))

    def fullspec(shape):
        nd = len(shape)
        return pl.BlockSpec(shape, lambda i: (0,) * nd)

    x0 = pl.pallas_call(
        _enc_kernel,
        grid=grid,
        in_specs=[
            pl.BlockSpec((BLK, 5), lambda i: (i, 0)),
            pl.BlockSpec((BLK, 3), lambda i: (i, 0)),
            pl.BlockSpec((BLK, 768), lambda i: (i, 0)),
            pl.BlockSpec((BLK, 768), lambda i: (i, 0)),
            pl.BlockSpec((BLK, 768), lambda i: (i, 0)),
            fullspec((5, DP)), fullspec((3, DP)), fullspec((768, DP)),
            fullspec((768, DP)), fullspec((768, DP)), fullspec((5, DP)),
            fullspec((DP, DP)), fullspec((3, DP)),
        ],
        out_specs=rowspec(),
        out_shape=jax.ShapeDtypeStruct((N, DP), f32),
    )(num_prop, cat_prop, tweet_emb, user_emb, user_name_emb,
      wn, wc, wt, wu, wun, pb, win, ones1)

    zeros_blk = jnp.zeros((K, DP), f32)

    def combine(x_arr, agg, root, wrel, brow):
        return pl.pallas_call(
            _combine_kernel,
            grid=grid,
            in_specs=[rowspec(),
                      pl.BlockSpec((NUM_REL, BLK, DP), lambda i: (0, i, 0)),
                      fullspec((DP, DP)), fullspec((NUM_REL, DP, DP)),
                      fullspec((1, DP))],
            out_specs=[rowspec(), fullspec((2, DP))],
            out_shape=[jax.ShapeDtypeStruct((N, DP), f32),
                       jax.ShapeDtypeStruct((2, DP), f32)],
            scratch_shapes=[pltpu.VMEM((2, DP), f32)],
        )(x_arr, agg, root, wrel, brow)

    def bn_apply(h, stats, gb):
        return pl.pallas_call(
            _bn_kernel,
            grid=grid,
            in_specs=[rowspec(), fullspec((2, DP)), fullspec((2, DP)),
                      fullspec((3, DP))],
            out_specs=rowspec(),
            out_shape=jax.ShapeDtypeStruct((N, DP), f32),
        )(h, stats, gb, ones1)

    agg1 = _edge_pass(x0, edata, zeros_blk)
    h1, stats1 = combine(x0, agg1, root1, w1, b1row)
    gb1 = jnp.stack([_pad1(p['bn1_g']), _pad1(p['bn1_b'])])
    x1 = bn_apply(h1, stats1, gb1)

    agg2 = _edge_pass(x1, edata, zeros_blk)
    h2, stats2 = combine(x1, agg2, root2, w2, b2row)
    gb2 = jnp.stack([_pad1(p['bn2_g']), _pad1(p['bn2_b'])])

    out = pl.pallas_call(
        _head_kernel,
        grid=grid,
        in_specs=[rowspec(), fullspec((2, DP)), fullspec((2, DP)),
                  fullspec((DP, DP)), fullspec((DP,)),
                  fullspec((DP, DP)), fullspec((DP,)),
                  fullspec((DP, DP)), fullspec((DP,))],
        out_specs=rowspec(),
        out_shape=jax.ShapeDtypeStruct((N, DP), f32),
    )(h2, stats2, gb2, _pad2(p['W_o1'], DP), _pad1(p['b_o1']),
      _pad2(p['W_o2'], DP), _pad1(p['b_o2']), _pad2(p['W_o3'], DP),
      _pad1(p['b_o3']))
    return out[:, :2]


# speedup vs baseline: 10.0454x; 1.0057x over previous
import jax
import jax.numpy as jnp
from jax import lax
from jax.experimental import pallas as pl
from jax.experimental.pallas import tpu as pltpu
from jax.experimental.pallas import tpu_sc as plsc

N = 50000
E = 800000
D = 100
NUM_REL = 3
DP = 128
ONES_LO = 112

CDST = 2560
NCHUNK = 20
NPAD = CDST * NCHUNK
ACC_ROWS = NUM_REL * CDST + 16
EPT = E // 16
EPT_PAD = 51200
EB = 2048
LCAP = 8192
K = 128
LROWS = LCAP // K

HI = lax.Precision.HIGHEST


def _leaky(x):
    return jnp.where(x > 0, x, 0.01 * x)


def _enc_kernel(num_ref, cat_ref, tw_ref, us_ref, un_ref,
                wn_ref, wc_ref, wt_ref, wu_ref, wun_ref, pb_ref,
                win_ref, ones_ref, x_ref):
    f32 = jnp.float32
    parts = []
    for j, (e_ref, w_ref) in enumerate(((num_ref, wn_ref), (cat_ref, wc_ref),
                                        (tw_ref, wt_ref), (us_ref, wu_ref),
                                        (un_ref, wun_ref))):
        part = _leaky(jnp.dot(e_ref[...], w_ref[...],
                              preferred_element_type=f32) + pb_ref[j])
        parts.append(part[:, :D // 5])
    parts.append(jnp.zeros((parts[0].shape[0], DP - D), f32))
    xcat = jnp.concatenate(parts, axis=1)
    x0 = _leaky(jnp.dot(xcat, win_ref[...], preferred_element_type=f32)
                + ones_ref[1][None, :])
    ones = ones_ref[0][None, :]
    x_ref[...] = x0 * (1.0 - ones) + ones


def _combine_kernel(x_ref, agg_ref, root_ref, wrel_ref, bias_ref,
                    h_ref, stats_ref, acc_ref):
    f32 = jnp.float32
    i = pl.program_id(0)
    h = jnp.dot(x_ref[...], root_ref[...], preferred_element_type=f32) \
        + bias_ref[0]
    for r in range(NUM_REL):
        blk = agg_ref[r]
        cnt = jnp.max(blk[:, ONES_LO:], axis=1, keepdims=True)
        mean = blk / jnp.maximum(cnt, 1.0)
        h = h + jnp.dot(mean, wrel_ref[r], preferred_element_type=f32)
    h_ref[...] = h

    @pl.when(i == 0)
    def _():
        acc_ref[...] = jnp.zeros_like(acc_ref)

    acc_ref[0, :] += jnp.sum(h, axis=0)
    acc_ref[1, :] += jnp.sum(h * h, axis=0)

    @pl.when(i == pl.num_programs(0) - 1)
    def _():
        stats_ref[...] = acc_ref[...]


def _bn(h, stats_ref, gb_ref):
    mean = stats_ref[0] * (1.0 / N)
    var = stats_ref[1] * (1.0 / N) - mean * mean
    scale = gb_ref[0] / jnp.sqrt(var + 1e-5)
    return _leaky((h - mean) * scale + gb_ref[1])


def _bn_kernel(h_ref, stats_ref, gb_ref, ones_ref, x_ref):
    x = _bn(h_ref[...], stats_ref, gb_ref)
    ones = ones_ref[0][None, :]
    x_ref[...] = x * (1.0 - ones) + ones


def _head_kernel(h_ref, stats_ref, gb_ref, w1_ref, b1_ref, w2_ref, b2_ref,
                 w3_ref, b3_ref, o_ref):
    f32 = jnp.float32
    x = _bn(h_ref[...], stats_ref, gb_ref)
    x = _leaky(jnp.dot(x, w1_ref[...], preferred_element_type=f32) + b1_ref[...])
    x = _leaky(jnp.dot(x, w2_ref[...], preferred_element_type=f32) + b2_ref[...])
    o_ref[...] = jnp.dot(x, w3_ref[...], preferred_element_type=f32) + b3_ref[...]


def _edge_pass_body(tab_hbm, edata_hbm, zeros_hbm,
                    agg_hbm,
                    e_a, e_b, glist, alist, buf, zbuf,
                    acc, sem, sem_e):
    cid = lax.axis_index("c")
    sid = lax.axis_index("s")
    e0 = sid * EPT
    trash = ACC_ROWS - 16 + sid
    lanes = lax.iota(jnp.int32, 16)
    lane15 = jnp.full((16,), 15, jnp.int32)
    NB = EPT_PAD // EB

    pltpu.sync_copy(zeros_hbm, zbuf)

    def stage(b, dst_ref):
        s0 = pl.multiple_of(b * EB, EB)
        return pltpu.async_copy(edata_hbm.at[sid, :, pl.ds(s0, EB)], dst_ref,
                                sem_e)

    def flush(j, done):
        row = (done // K) & (LROWS - 1)
        pltpu.async_copy(tab_hbm.at[glist.at[row]], buf, sem).wait()
        pltpu.sync_copy(buf, acc.at[alist.at[row]], add=True)
        return done + K

    def chunk_body(k, _unused):
        base_dst = (cid * (NCHUNK // 2) + k) * CDST
        z0 = sid * (ACC_ROWS // 16)
        for off in range(0, ACC_ROWS // 16, K):
            ln = min(K, ACC_ROWS // 16 - off)
            pltpu.sync_copy(zbuf.at[pl.ds(0, ln)], acc.at[pl.ds(z0 + off, ln)])
        plsc.subcore_barrier()

        def scan(e_ref, carry):
            off_s, done = carry

            @plsc.parallel_loop(0, EB // 16, unroll=8,
                                carry=jnp.full((16,), off_s, jnp.int32))
            def vit(i, offv):
                ii = pl.multiple_of(i * 16, 16)
                s = e_ref[0, pl.ds(ii, 16)]
                d = e_ref[1, pl.ds(ii, 16)]
                r = e_ref[2, pl.ds(ii, 16)]
                dl = d - base_dst
                m = (dl >= 0) & (dl < CDST)
                ai = r * CDST + dl
                pos = offv + plsc.cumsum(m.astype(jnp.int32)) - 1
                prow = (pos // K) & (LROWS - 1)
                pcol = pos & (K - 1)
                plsc.store_scatter(glist, [prow, pcol], s, mask=m)
                plsc.store_scatter(alist, [prow, pcol], ai, mask=m)
                return pos.at[lane15].get(mode="promise_in_bounds") + 1

            off_s2 = jnp.max(vit)
            nb = (off_s2 - done) // K
            done2 = lax.fori_loop(0, nb, flush, done)
            return off_s2, done2

        stage(0, e_a).wait()

        def pair(j, carry):
            cp_b = stage(2 * j + 1, e_b)
            carry = scan(e_a, carry)
            cp_b.wait()
            cp_a = stage(2 * j + 2, e_a)
            carry = scan(e_b, carry)
            cp_a.wait()
            return carry

        carry = lax.fori_loop(0, (NB - 1) // 2, pair,
                              (jnp.int32(0), jnp.int32(0)))
        off_s, done = scan(e_a, carry)
        rem = off_s - done

        def padfill(i, _):
            pos = off_s + i * 16 + lanes
            m = pos < done + K
            prow = (pos // K) & (LROWS - 1)
            pcol = pos & (K - 1)
            plsc.store_scatter(glist, [prow, pcol],
                               jnp.full((16,), sid * 64, jnp.int32), mask=m)
            plsc.store_scatter(alist, [prow, pcol],
                               jnp.full((16,), trash, jnp.int32), mask=m)
            return 0

        @pl.when(rem > 0)
        def _():
            lax.fori_loop(0, K // 16, padfill, 0)
            flush(0, done)

        plsc.subcore_barrier()
        for r in range(NUM_REL):
            pltpu.sync_copy(
                acc.at[pl.ds(r * CDST + sid * (CDST // 16), CDST // 16)],
                agg_hbm.at[r, pl.ds(base_dst + sid * (CDST // 16), CDST // 16)])
        plsc.subcore_barrier()
        return 0

    lax.fori_loop(0, NCHUNK // 2, chunk_body, 0)


def _edge_pass(table, edata, zeros_blk):
    mesh = plsc.VectorSubcoreMesh(core_axis_name="c", subcore_axis_name="s")
    f = pl.kernel(
        _edge_pass_body,
        mesh=mesh,
        compiler_params=pltpu.CompilerParams(needs_layout_passes=False),
        out_type=jax.ShapeDtypeStruct((NUM_REL, NPAD, DP), jnp.float32),
        scratch_types=[
            pltpu.VMEM((3, EB), jnp.int32),
            pltpu.VMEM((3, EB), jnp.int32),
            pltpu.VMEM((LROWS, K), jnp.int32),
            pltpu.VMEM((LROWS, K), jnp.int32),
            pltpu.VMEM((K, DP), jnp.float32),
            pltpu.VMEM((K, DP), jnp.float32),
            pltpu.VMEM_SHARED((ACC_ROWS, DP), jnp.float32),
            pltpu.SemaphoreType.DMA,
            pltpu.SemaphoreType.DMA,
        ],
    )
    return f(table, edata, zeros_blk)


def _pad2(w, rows, cols=DP):
    out = jnp.zeros((rows, cols), jnp.float32)
    return out.at[:w.shape[0], :w.shape[1]].set(w)


def _pad1(b, cols=DP):
    return jnp.zeros((cols,), jnp.float32).at[:b.shape[0]].set(b)


def kernel(num_prop, cat_prop, tweet_emb, user_emb, user_name_emb,
           edge_index, edge_type, params):
    p = params
    f32 = jnp.float32
    edata = jnp.concatenate([edge_index.astype(jnp.int32),
                             edge_type.astype(jnp.int32)[None, :]], axis=0)
    edata = edata.reshape(3, 16, EPT)
    pad = jnp.zeros((3, 16, EPT_PAD - EPT), jnp.int32).at[1].set(1 << 20)
    edata = jnp.concatenate([edata, pad], axis=2).transpose(1, 0, 2)

    IND = D // 5
    wn = _pad2(p['W_num'], 5)
    wc = _pad2(p['W_cat'], 3)
    wt = _pad2(p['W_tweet'], 768)
    wu = _pad2(p['W_user'], 768)
    wun = _pad2(p['W_uname'], 768)
    pb = jnp.stack([_pad1(p['b_num']), _pad1(p['b_cat']), _pad1(p['b_tweet']),
                    _pad1(p['b_user']), _pad1(p['b_uname'])])
    win = _pad2(p['W_in'], DP)
    ones1 = jnp.zeros((3, DP), f32)
    ones1 = ones1.at[0, ONES_LO:].set(1.0)
    ones1 = ones1.at[1, :D].set(p['b_in'])

    root1 = _pad2(p['rgcn1_root'], DP)
    w1 = jnp.stack([_pad2(p['rgcn1_W'][r], DP) for r in range(NUM_REL)])
    b1row = _pad1(p['rgcn1_b'])[None, :]
    root2 = _pad2(p['rgcn2_root'], DP)
    w2 = jnp.stack([_pad2(p['rgcn2_W'][r], DP) for r in range(NUM_REL)])
    b2row = _pad1(p['rgcn2_b'])[None, :]

    BLK = 1000
    grid = (N // BLK,)

    def rowspec(d2=DP):
        return pl.BlockSpec((BLK, d2), lambda i: (i, 0))

    def fullspec(shape):
        nd = len(shape)
        return pl.BlockSpec(shape, lambda i: (0,) * nd)

    x0 = pl.pallas_call(
        _enc_kernel,
        grid=grid,
        in_specs=[
            pl.BlockSpec((BLK, 5), lambda i: (i, 0)),
            pl.BlockSpec((BLK, 3), lambda i: (i, 0)),
            pl.BlockSpec((BLK, 768), lambda i: (i, 0)),
            pl.BlockSpec((BLK, 768), lambda i: (i, 0)),
            pl.BlockSpec((BLK, 768), lambda i: (i, 0)),
            fullspec((5, DP)), fullspec((3, DP)), fullspec((768, DP)),
            fullspec((768, DP)), fullspec((768, DP)), fullspec((5, DP)),
            fullspec((DP, DP)), fullspec((3, DP)),
        ],
        out_specs=rowspec(),
        out_shape=jax.ShapeDtypeStruct((N, DP), f32),
    )(num_prop, cat_prop, tweet_emb, user_emb, user_name_emb,
      wn, wc, wt, wu, wun, pb, win, ones1)

    zeros_blk = jnp.zeros((K, DP), f32)

    def combine(x_arr, agg, root, wrel, brow):
        return pl.pallas_call(
            _combine_kernel,
            grid=grid,
            in_specs=[rowspec(),
                      pl.BlockSpec((NUM_REL, BLK, DP), lambda i: (0, i, 0)),
                      fullspec((DP, DP)), fullspec((NUM_REL, DP, DP)),
                      fullspec((1, DP))],
            out_specs=[rowspec(), fullspec((2, DP))],
            out_shape=[jax.ShapeDtypeStruct((N, DP), f32),
                       jax.ShapeDtypeStruct((2, DP), f32)],
            scratch_shapes=[pltpu.VMEM((2, DP), f32)],
        )(x_arr, agg, root, wrel, brow)

    def bn_apply(h, stats, gb):
        return pl.pallas_call(
            _bn_kernel,
            grid=grid,
            in_specs=[rowspec(), fullspec((2, DP)), fullspec((2, DP)),
                      fullspec((3, DP))],
            out_specs=rowspec(),
            out_shape=jax.ShapeDtypeStruct((N, DP), f32),
        )(h, stats, gb, ones1)

    agg1 = _edge_pass(x0, edata, zeros_blk)
    h1, stats1 = combine(x0, agg1, root1, w1, b1row)
    gb1 = jnp.stack([_pad1(p['bn1_g']), _pad1(p['bn1_b'])])
    x1 = bn_apply(h1, stats1, gb1)

    agg2 = _edge_pass(x1, edata, zeros_blk)
    h2, stats2 = combine(x1, agg2, root2, w2, b2row)
    gb2 = jnp.stack([_pad1(p['bn2_g']), _pad1(p['bn2_b'])])

    out = pl.pallas_call(
        _head_kernel,
        grid=grid,
        in_specs=[rowspec(), fullspec((2, DP)), fullspec((2, DP)),
                  fullspec((DP, DP)), fullspec((DP,)),
                  fullspec((DP, DP)), fullspec((DP,)),
                  fullspec((DP, DP)), fullspec((DP,))],
        out_specs=rowspec(),
        out_shape=jax.ShapeDtypeStruct((N, DP), f32),
    )(h2, stats2, gb2, _pad2(p['W_o1'], DP), _pad1(p['b_o1']),
      _pad2(p['W_o2'], DP), _pad1(p['b_o2']), _pad2(p['W_o3'], DP),
      _pad1(p['b_o3']))
    return out[:, :2]
